# Initial kernel scaffold; baseline (speedup 1.0000x reference)
import jax
import jax.numpy as jnp
from jax.experimental import pallas as pl


def kernel(x, edge_index, W0, b0, W1, b1, W2, b2, W3, b3, W4, b4, W5, b5, W6, b6):
    raise NotImplementedError("write your pallas kernel here")


# speedup vs baseline: 6.0046x; 6.0046x over previous
import functools

import jax
import jax.numpy as jnp
from jax import lax
from jax.experimental import pallas as pl
from jax.experimental.pallas import tpu as pltpu
from jax.experimental.pallas import tpu_sc as plsc

NC = 2
NS = 16
NW = NC * NS
CHUNK = 128
NBUF = 2
BLK = 1264


@functools.lru_cache(maxsize=None)
def _seg_sum_kernel(np_, na, k, d):
    mesh = plsc.VectorSubcoreMesh(core_axis_name="c", subcore_axis_name="s", num_cores=NC, num_subcores=NS)
    rpt = -(-(na // NS) // 8) * 8
    rlast = na - rpt * (NS - 1)

    @functools.partial(
        pl.kernel,
        out_type=jax.ShapeDtypeStruct((NC, np_, d), jnp.float32),
        mesh=mesh,
        scratch_types=(
            [pltpu.VMEM((k, CHUNK), jnp.int32)]
            + [pltpu.VMEM((CHUNK, d), jnp.float32)] * NBUF
            + [pltpu.VMEM((CHUNK,), jnp.int32)] * NBUF
            + [pltpu.VMEM_SHARED((na, d), jnp.float32)]
            + [pltpu.SemaphoreType.DMA] * (3 * NBUF)
        ),
    )
    def seg(hp_hbm, src_hbm, dst_hbm, z_hbm, out_hbm, *scr):
        src_v = scr[0]
        bufs = scr[1:1 + NBUF]
        dbufs = scr[1 + NBUF:1 + 2 * NBUF]
        acc = scr[1 + 2 * NBUF]
        sems = scr[2 + 2 * NBUF:]
        gsems = sems[:NBUF]
        ssems = sems[NBUF:2 * NBUF]
        isems = sems[2 * NBUF:]
        c = lax.axis_index("c")
        s = lax.axis_index("s")
        wid = s * NC + c
        src_w = src_hbm.at[wid]
        dst_w = dst_hbm.at[wid]
        pltpu.sync_copy(src_w, src_v)

        @pl.when(s < NS - 1)
        def _():
            pltpu.sync_copy(z_hbm.at[pl.ds(s * rpt, rpt)], acc.at[pl.ds(s * rpt, rpt)])

        @pl.when(s == NS - 1)
        def _():
            pltpu.sync_copy(z_hbm.at[pl.ds((NS - 1) * rpt, rlast)],
                            acc.at[pl.ds((NS - 1) * rpt, rlast)])

        plsc.subcore_barrier()

        for b in range(NBUF):
            pltpu.async_copy(hp_hbm.at[src_v.at[b]], bufs[b], gsems[b])
            pltpu.async_copy(dst_w.at[b], dbufs[b], isems[b])

        def step(i, carry):
            j = i * NBUF
            for b in range(NBUF):
                jj = j + b
                pltpu.make_async_copy(hp_hbm.at[src_v.at[jj]], bufs[b], gsems[b]).wait()
                pltpu.make_async_copy(dst_w.at[jj], dbufs[b], isems[b]).wait()
                pltpu.async_copy(bufs[b], acc.at[dbufs[b]], ssems[b], add=True)
            for b in range(NBUF):
                jj = j + b
                pltpu.make_async_copy(bufs[b], acc.at[dbufs[b]], ssems[b]).wait()
                nj = jj + NBUF

                @pl.when(nj < k)
                def _():
                    pltpu.async_copy(hp_hbm.at[src_v.at[nj]], bufs[b], gsems[b])
                    pltpu.async_copy(dst_w.at[nj], dbufs[b], isems[b])
            return carry

        lax.fori_loop(0, k // NBUF, step, 0)
        plsc.subcore_barrier()

        @pl.when(s < NS - 1)
        def _():
            pltpu.sync_copy(acc.at[pl.ds(s * rpt, rpt)],
                            out_hbm.at[c].at[pl.ds(s * rpt, rpt)])

        @pl.when(s == NS - 1)
        def _():
            pltpu.sync_copy(acc.at[pl.ds((NS - 1) * rpt, rlast)],
                            out_hbm.at[c].at[pl.ds((NS - 1) * rpt, rlast)])

    return seg


def _seg_sum_call(hp, src3, dst3, zrows, np_, na, k, d):
    return _seg_sum_kernel(np_, na, k, d)(hp, src3, dst3, zrows)


def _deg_call(dst3, ones_d, zrows, np_, na, k, d):
    mesh = plsc.VectorSubcoreMesh(core_axis_name="c", subcore_axis_name="s", num_cores=NC, num_subcores=NS)
    rpt = -(-(na // NS) // 8) * 8
    rlast = na - rpt * (NS - 1)

    @functools.partial(
        pl.kernel,
        out_type=jax.ShapeDtypeStruct((NC, np_, d), jnp.float32),
        mesh=mesh,
        scratch_types=(
            [pltpu.VMEM((CHUNK,), jnp.int32)] * NBUF
            + [pltpu.VMEM((CHUNK, d), jnp.float32)]
            + [pltpu.VMEM_SHARED((na, d), jnp.float32)]
            + [pltpu.SemaphoreType.DMA] * (2 * NBUF)
        ),
    )
    def deg(dst_hbm, ones_hbm, z_hbm, out_hbm, *scr):
        dbufs = scr[:NBUF]
        ones_v = scr[NBUF]
        acc = scr[NBUF + 1]
        sems = scr[NBUF + 2:]
        ssems = sems[:NBUF]
        isems = sems[NBUF:]
        c = lax.axis_index("c")
        s = lax.axis_index("s")
        wid = s * NC + c
        dst_w = dst_hbm.at[wid]
        pltpu.sync_copy(ones_hbm, ones_v)
        @pl.when(s < NS - 1)
        def _():
            pltpu.sync_copy(z_hbm.at[pl.ds(s * rpt, rpt)], acc.at[pl.ds(s * rpt, rpt)])

        @pl.when(s == NS - 1)
        def _():
            pltpu.sync_copy(z_hbm.at[pl.ds((NS - 1) * rpt, rlast)],
                            acc.at[pl.ds((NS - 1) * rpt, rlast)])

        plsc.subcore_barrier()

        for b in range(NBUF):
            pltpu.async_copy(dst_w.at[b], dbufs[b], isems[b])

        def step(i, carry):
            j = i * NBUF
            for b in range(NBUF):
                jj = j + b
                pltpu.make_async_copy(dst_w.at[jj], dbufs[b], isems[b]).wait()
                pltpu.async_copy(ones_v, acc.at[dbufs[b]], ssems[b], add=True)
            for b in range(NBUF):
                jj = j + b
                pltpu.make_async_copy(ones_v, acc.at[dbufs[b]], ssems[b]).wait()
                nj = jj + NBUF

                @pl.when(nj < k)
                def _():
                    pltpu.async_copy(dst_w.at[nj], dbufs[b], isems[b])
            return carry

        lax.fori_loop(0, k // NBUF, step, 0)
        plsc.subcore_barrier()

        @pl.when(s < NS - 1)
        def _():
            pltpu.sync_copy(acc.at[pl.ds(s * rpt, rpt)],
                            out_hbm.at[c].at[pl.ds(s * rpt, rpt)])

        @pl.when(s == NS - 1)
        def _():
            pltpu.sync_copy(acc.at[pl.ds((NS - 1) * rpt, rlast)],
                            out_hbm.at[c].at[pl.ds((NS - 1) * rpt, rlast)])

    return deg(dst3, ones_d, zrows)


def _pre_body(x_ref, deg_ref, w_ref, hp_ref, dv_ref):
    deg = deg_ref[0, :, :1] + deg_ref[1, :, :1] + 1.0
    dinv = lax.rsqrt(deg)
    h = jnp.dot(x_ref[...], w_ref[...], preferred_element_type=jnp.float32)
    hp_ref[...] = h * dinv
    dv_ref[...] = jnp.broadcast_to(dinv, dv_ref.shape)


def _tc_pre(xp, deg2, w0):
    np_, d = xp.shape
    grid = (np_ // BLK,)
    return pl.pallas_call(
        _pre_body,
        grid=grid,
        in_specs=[
            pl.BlockSpec((BLK, d), lambda i: (i, 0)),
            pl.BlockSpec((NC, BLK, d), lambda i: (0, i, 0)),
            pl.BlockSpec((d, d), lambda i: (0, 0)),
        ],
        out_specs=[
            pl.BlockSpec((BLK, d), lambda i: (i, 0)),
            pl.BlockSpec((BLK, 16), lambda i: (i, 0)),
        ],
        out_shape=[
            jax.ShapeDtypeStruct((np_, d), jnp.float32),
            jax.ShapeDtypeStruct((np_, 16), jnp.float32),
        ],
    )(xp, deg2, w0)


def _tc_mid(s2, hp, dinv16, b_prev, w, skip=None, w_skip=None, emit_a=False):
    np_, d = hp.shape
    grid = (np_ // BLK,)
    has_skip = skip is not None

    def body(*refs):
        s_ref, hp_ref, dv_ref, b_ref, w_ref = refs[:5]
        rest = refs[5:]
        if has_skip:
            skip_ref, wskip_ref = rest[0], rest[1]
            rest = rest[2:]
        dinv = dv_ref[:, :1]
        a = dinv * (s_ref[0] + s_ref[1] + hp_ref[...]) + b_ref[...]
        a = jnp.maximum(a, 0.0)
        h = jnp.dot(a, w_ref[...], preferred_element_type=jnp.float32)
        if has_skip:
            h = h + jnp.dot(skip_ref[...], wskip_ref[...],
                            preferred_element_type=jnp.float32)
        rest[0][...] = dinv * h
        if emit_a:
            rest[1][...] = a

    in_specs = [
        pl.BlockSpec((NC, BLK, d), lambda i: (0, i, 0)),
        pl.BlockSpec((BLK, d), lambda i: (i, 0)),
        pl.BlockSpec((BLK, 16), lambda i: (i, 0)),
        pl.BlockSpec((1, d), lambda i: (0, 0)),
        pl.BlockSpec((d, d), lambda i: (0, 0)),
    ]
    args = [s2, hp, dinv16, b_prev, w]
    if has_skip:
        in_specs += [pl.BlockSpec((BLK, d), lambda i: (i, 0)),
                     pl.BlockSpec((d, d), lambda i: (0, 0))]
        args += [skip, w_skip]
    out_specs = [pl.BlockSpec((BLK, d), lambda i: (i, 0))]
    out_shape = [jax.ShapeDtypeStruct((np_, d), jnp.float32)]
    if emit_a:
        out_specs.append(pl.BlockSpec((BLK, d), lambda i: (i, 0)))
        out_shape.append(jax.ShapeDtypeStruct((np_, d), jnp.float32))
    res = pl.pallas_call(
        body, grid=grid, in_specs=in_specs, out_specs=out_specs,
        out_shape=out_shape,
    )(*args)
    return res if emit_a else res[0]


def _final_body(s_ref, hp_ref, dv_ref, b_ref, out_ref):
    dinv = dv_ref[:, :1]
    out_ref[...] = dinv * (s_ref[0] + s_ref[1] + hp_ref[...]) + b_ref[...]


def _tc_final(s2, hp, dinv16, b):
    np_, d = hp.shape
    grid = (np_ // BLK,)
    return pl.pallas_call(
        _final_body,
        grid=grid,
        in_specs=[
            pl.BlockSpec((NC, BLK, d), lambda i: (0, i, 0)),
            pl.BlockSpec((BLK, d), lambda i: (i, 0)),
            pl.BlockSpec((BLK, 16), lambda i: (i, 0)),
            pl.BlockSpec((1, d), lambda i: (0, 0)),
        ],
        out_specs=pl.BlockSpec((BLK, d), lambda i: (i, 0)),
        out_shape=jax.ShapeDtypeStruct((np_, d), jnp.float32),
    )(s2, hp, dinv16, b)


def kernel(x, edge_index, W0, b0, W1, b1, W2, b2, W3, b3, W4, b4, W5, b5, W6, b6):
    n, d = x.shape
    e = edge_index.shape[1]

    per_tile = -(-e // (NW * CHUNK * NBUF)) * (CHUNK * NBUF)
    k = per_tile // CHUNK
    e_pad = NW * per_tile
    np_ = -(-(n + 1) // BLK) * BLK
    na = -(-(n + 1) // NS) * NS
    pad_e = e_pad - e
    src3 = jnp.concatenate(
        [edge_index[0], jnp.zeros((pad_e,), edge_index.dtype)]).reshape(NW, k, CHUNK)
    dst3 = jnp.concatenate(
        [edge_index[1], jnp.full((pad_e,), n, edge_index.dtype)]).reshape(NW, k, CHUNK)
    xp = jnp.pad(x, ((0, np_ - n), (0, 0)))
    zrows = jnp.zeros((np_, d), jnp.float32)
    ones_d = jnp.ones((CHUNK, d), jnp.float32)
    b0r, b1r, b2r, b3r, b4r, b5r, b6r = (
        bb.reshape(1, d) for bb in (b0, b1, b2, b3, b4, b5, b6))

    deg2 = _deg_call(dst3, ones_d, zrows, np_, na, k, d)
    hp0, dinv16 = _tc_pre(xp, deg2, W0)
    s = _seg_sum_call(hp0, src3, dst3, zrows, np_, na, k, d)
    hp1 = _tc_mid(s, hp0, dinv16, b0r, W1)
    s = _seg_sum_call(hp1, src3, dst3, zrows, np_, na, k, d)
    hp2, a1 = _tc_mid(s, hp1, dinv16, b1r, W2, emit_a=True)
    s = _seg_sum_call(hp2, src3, dst3, zrows, np_, na, k, d)
    hp3 = _tc_mid(s, hp2, dinv16, b2r, W3)
    s = _seg_sum_call(hp3, src3, dst3, zrows, np_, na, k, d)
    hp4 = _tc_mid(s, hp3, dinv16, b3r, W4[:d], skip=a1, w_skip=W4[d:])
    s = _seg_sum_call(hp4, src3, dst3, zrows, np_, na, k, d)
    hp5 = _tc_mid(s, hp4, dinv16, b4r, W5)
    s = _seg_sum_call(hp5, src3, dst3, zrows, np_, na, k, d)
    hp6 = _tc_mid(s, hp5, dinv16, b5r, W6[:d], skip=xp, w_skip=W6[d:])
    s = _seg_sum_call(hp6, src3, dst3, zrows, np_, na, k, d)
    out = _tc_final(s, hp6, dinv16, b6r)
    return out[:n]
